# baseline (speedup 1.0000x reference)
import jax
import jax.numpy as jnp
from jax import lax
from jax.experimental import pallas as pl
from jax.experimental.pallas import tpu as pltpu

L = 128
H = 512
V = 8192
BEAM = 8
NB = 8
MAXLEN = 8
NEGINF = float("-inf")
BIG = 1 << 30


def _topk8_body(c_ref, v_ref, i_ref):
    f32 = jnp.float32
    i32 = jnp.int32
    n = c_ref.shape[1]
    w = 1024
    nc = n // w
    lane8 = lax.broadcasted_iota(i32, (8, 8), 1)
    lane_w = lax.broadcasted_iota(i32, (8, w), 1)
    iota = lax.broadcasted_iota(i32, (8, n), 1)
    sbm = c_ref[...]
    vals = jnp.zeros((8, 8), f32)
    idxs = jnp.zeros((8, 8), i32)
    for k in range(0, 8, 2):
        m1 = jnp.full((8, w), NEGINF, f32)
        c1 = jnp.zeros((8, w), i32)
        m2 = jnp.full((8, w), NEGINF, f32)
        c2 = jnp.zeros((8, w), i32)
        for c in range(nc):
            blk = sbm[:, c * w:(c + 1) * w]
            u1 = blk > m1
            dv = jnp.where(u1, m1, blk)
            dc = jnp.where(u1, c1, c)
            c1 = jnp.where(u1, c, c1)
            m1 = jnp.where(u1, blk, m1)
            u2 = (dv > m2) | ((dv == m2) & (dc < c2))
            c2 = jnp.where(u2, dc, c2)
            m2 = jnp.where(u2, dv, m2)
        mx = jnp.max(m1, axis=1, keepdims=True)
        pos = jnp.min(jnp.where(m1 == mx, c1 * w + lane_w, BIG),
                      axis=1, keepdims=True)
        vals = jnp.where(lane8 == k, mx, vals)
        idxs = jnp.where(lane8 == k, pos, idxs)
        lam = pos - (pos // w) * w
        cm = jnp.where(lane_w == lam, m2, m1)
        cc = jnp.where(lane_w == lam, c2, c1)
        mx2 = jnp.max(cm, axis=1, keepdims=True)
        pos2 = jnp.min(jnp.where(cm == mx2, cc * w + lane_w, BIG),
                       axis=1, keepdims=True)
        vals = jnp.where(lane8 == k + 1, mx2, vals)
        idxs = jnp.where(lane8 == k + 1, pos2, idxs)
        if k < 6:
            sbm = jnp.where((iota == pos) | (iota == pos2), NEGINF, sbm)
    v_ref[...] = vals
    i_ref[...] = idxs


def _topk8(flat):
    return pl.pallas_call(
        _topk8_body,
        out_shape=[
            jax.ShapeDtypeStruct((8, 8), jnp.float32),
            jax.ShapeDtypeStruct((8, 8), jnp.int32),
        ],
        in_specs=[pl.BlockSpec(memory_space=pltpu.VMEM)],
        out_specs=[
            pl.BlockSpec(memory_space=pltpu.VMEM),
            pl.BlockSpec(memory_space=pltpu.VMEM),
        ],
    )(flat)


def _decode(hidden, tok, enc, mask, emb, W_att, W_in, W_hh, W_out, b_out):
    x = jnp.take(emb, tok, axis=0)
    q = hidden @ W_att
    scores = jnp.einsum('nh,nlh->nl', q, enc) / jnp.sqrt(jnp.float32(H))
    scores = jnp.where(mask, scores, -1e9)
    a = jax.nn.softmax(scores, axis=-1)
    ctx = jnp.einsum('nl,nlh->nh', a, enc)
    new_h = jnp.tanh(x @ W_in + hidden @ W_hh + ctx)
    logits = new_h @ W_out + b_out
    logp = jax.nn.log_softmax(logits, axis=-1)
    return new_h, logp


def kernel(c_encoder_outputs, c_encoder_inputs_length, h_encoder_outputs,
           decoder_hidden_state, decoder_input, batch_size, beam_width,
           best_n, eosid, max_len, emb, W_att, W_in, W_hh, W_out, b_out):
    enc = jnp.transpose(c_encoder_outputs, (1, 0, 2))
    if h_encoder_outputs is not None:
        enc = enc + jnp.transpose(h_encoder_outputs, (1, 0, 2))
    Bc = c_encoder_outputs.shape[1]
    lengths = c_encoder_inputs_length[0]
    mask = jnp.arange(L)[None, :] < jnp.maximum(lengths, 1)[:, None]
    hidden = decoder_hidden_state[-1]
    tok = decoder_input[0]

    new_h, logp = _decode(hidden, tok, enc, mask, emb, W_att, W_in, W_hh,
                          W_out, b_out)
    beam_scores, tok_b = _topk8(logp)
    beam_scores = beam_scores + jnp.asarray(
        (batch_size - Bc) + (beam_width - BEAM) + (best_n - 4), jnp.float32)
    hidden_b = jnp.broadcast_to(new_h[:, None, :], (Bc, BEAM, H))
    seqs = tok_b[:, :, None]
    finished = tok_b == eosid

    enc_b = jnp.broadcast_to(enc[:, None], (Bc, BEAM, L, H)).reshape(Bc * BEAM, L, H)
    mask_b = jnp.broadcast_to(mask[:, None], (Bc, BEAM, L)).reshape(Bc * BEAM, L)
    eos_row = jnp.where(jnp.arange(V) == eosid, 0.0, -1e9)

    for _ in range(1, MAXLEN):
        h_flat = hidden_b.reshape(-1, H)
        t_flat = tok_b.reshape(-1)
        new_h, logp = _decode(h_flat, t_flat, enc_b, mask_b, emb, W_att,
                              W_in, W_hh, W_out, b_out)
        logp = logp.reshape(Bc, BEAM, V)
        add = jnp.where(finished[:, :, None], eos_row[None, None, :], logp)
        cand = beam_scores[:, :, None] + add
        flat = cand.reshape(Bc, BEAM * V)
        beam_scores, flat_idx = _topk8(flat)
        src = flat_idx // V
        tok_b = flat_idx % V
        new_h = new_h.reshape(Bc, BEAM, H)
        hidden_b = jnp.take_along_axis(new_h, src[:, :, None], axis=1)
        seqs = jnp.take_along_axis(seqs, src[:, :, None], axis=1)
        seqs = jnp.concatenate([seqs, tok_b[:, :, None]], axis=2)
        finished = jnp.take_along_axis(finished, src, axis=1) | (tok_b == eosid)

    final_scores = beam_scores / jnp.asarray(MAXLEN - 1 + 1e-6, jnp.float32)
    return final_scores[:, :4], seqs[:, :4, :]


# speedup vs baseline: 1.0192x; 1.0192x over previous
import jax
import jax.numpy as jnp
from jax import lax
from jax.experimental import pallas as pl
from jax.experimental.pallas import tpu as pltpu

L = 128
H = 512
V = 8192
BEAM = 8
NB = 8
MAXLEN = 8
NEGINF = float("-inf")
BIG = 1 << 30


def _topk8_body(c_ref, v_ref, i_ref):
    f32 = jnp.float32
    i32 = jnp.int32
    n = c_ref.shape[1]
    w = 1024
    nc = n // w
    lane8 = lax.broadcasted_iota(i32, (8, 8), 1)
    lane_w = lax.broadcasted_iota(i32, (8, w), 1)
    iota = lax.broadcasted_iota(i32, (8, n), 1)
    sbm = c_ref[...]
    vals = jnp.zeros((8, 8), f32)
    idxs = jnp.zeros((8, 8), i32)
    for k in range(8):
        colmax = jnp.full((8, w), NEGINF, f32)
        colchunk = jnp.zeros((8, w), i32)
        for c in range(nc):
            blk = sbm[:, c * w:(c + 1) * w]
            upd = blk > colmax
            colchunk = jnp.where(upd, c, colchunk)
            colmax = jnp.where(upd, blk, colmax)
        mx = jnp.max(colmax, axis=1, keepdims=True)
        pos = jnp.min(jnp.where(colmax == mx, colchunk * w + lane_w, BIG),
                      axis=1, keepdims=True)
        vals = jnp.where(lane8 == k, mx, vals)
        idxs = jnp.where(lane8 == k, pos, idxs)
        if k < 7:
            sbm = jnp.where(iota == pos, NEGINF, sbm)
    v_ref[...] = vals
    i_ref[...] = idxs


def _topk8(flat):
    return pl.pallas_call(
        _topk8_body,
        out_shape=[
            jax.ShapeDtypeStruct((8, 8), jnp.float32),
            jax.ShapeDtypeStruct((8, 8), jnp.int32),
        ],
        in_specs=[pl.BlockSpec(memory_space=pltpu.VMEM)],
        out_specs=[
            pl.BlockSpec(memory_space=pltpu.VMEM),
            pl.BlockSpec(memory_space=pltpu.VMEM),
        ],
    )(flat)


def _decode(hidden, tok, enc, mask, emb, W_att, W_in, W_hh, W_out, b_out):
    x = jnp.take(emb, tok, axis=0)
    q = hidden @ W_att
    scores = jnp.einsum('nh,nlh->nl', q, enc) / jnp.sqrt(jnp.float32(H))
    scores = jnp.where(mask, scores, -1e9)
    a = jax.nn.softmax(scores, axis=-1)
    ctx = jnp.einsum('nl,nlh->nh', a, enc)
    new_h = jnp.tanh(x @ W_in + hidden @ W_hh + ctx)
    logits = new_h @ W_out + b_out
    logp = jax.nn.log_softmax(logits, axis=-1)
    return new_h, logp


def kernel(c_encoder_outputs, c_encoder_inputs_length, h_encoder_outputs,
           decoder_hidden_state, decoder_input, batch_size, beam_width,
           best_n, eosid, max_len, emb, W_att, W_in, W_hh, W_out, b_out):
    enc = jnp.transpose(c_encoder_outputs, (1, 0, 2))
    if h_encoder_outputs is not None:
        enc = enc + jnp.transpose(h_encoder_outputs, (1, 0, 2))
    Bc = c_encoder_outputs.shape[1]
    lengths = c_encoder_inputs_length[0]
    mask = jnp.arange(L)[None, :] < jnp.maximum(lengths, 1)[:, None]
    hidden = decoder_hidden_state[-1]
    tok = decoder_input[0]

    new_h, logp = _decode(hidden, tok, enc, mask, emb, W_att, W_in, W_hh,
                          W_out, b_out)
    beam_scores, tok_b = _topk8(logp)
    beam_scores = beam_scores + jnp.asarray(
        (batch_size - Bc) + (beam_width - BEAM) + (best_n - 4), jnp.float32)
    hidden_b = jnp.broadcast_to(new_h[:, None, :], (Bc, BEAM, H))
    seqs = tok_b[:, :, None]
    finished = tok_b == eosid

    enc_b = jnp.broadcast_to(enc[:, None], (Bc, BEAM, L, H)).reshape(Bc * BEAM, L, H)
    mask_b = jnp.broadcast_to(mask[:, None], (Bc, BEAM, L)).reshape(Bc * BEAM, L)
    eos_row = jnp.where(jnp.arange(V) == eosid, 0.0, -1e9)

    for _ in range(1, MAXLEN):
        h_flat = hidden_b.reshape(-1, H)
        t_flat = tok_b.reshape(-1)
        new_h, logp = _decode(h_flat, t_flat, enc_b, mask_b, emb, W_att,
                              W_in, W_hh, W_out, b_out)
        logp = logp.reshape(Bc, BEAM, V)
        add = jnp.where(finished[:, :, None], eos_row[None, None, :], logp)
        cand = beam_scores[:, :, None] + add
        flat = cand.reshape(Bc, BEAM * V)
        beam_scores, flat_idx = _topk8(flat)
        src = flat_idx // V
        tok_b = flat_idx % V
        new_h = new_h.reshape(Bc, BEAM, H)
        hidden_b = jnp.take_along_axis(new_h, src[:, :, None], axis=1)
        seqs = jnp.take_along_axis(seqs, src[:, :, None], axis=1)
        seqs = jnp.concatenate([seqs, tok_b[:, :, None]], axis=2)
        finished = jnp.take_along_axis(finished, src, axis=1) | (tok_b == eosid)

    final_scores = beam_scores / jnp.asarray(MAXLEN - 1 + 1e-6, jnp.float32)
    return final_scores[:, :4], seqs[:, :4, :]
